# Initial kernel scaffold; baseline (speedup 1.0000x reference)
import jax
import jax.numpy as jnp
from jax.experimental import pallas as pl


def kernel(node_feat, edge_feat, edge_index, in_w, out_w, src_w, dst_w, nloop_w, eloop_w, nbias, ebias, nw1, nb1, ngamma, nbeta, nw2, nb2, ew1, eb1, egamma, ebeta, ew2, eb2):
    raise NotImplementedError("write your pallas kernel here")


# speedup vs baseline: 1.9561x; 1.9561x over previous
import dataclasses
import functools

import jax
import jax.numpy as jnp
from jax import lax
from jax.experimental import pallas as pl
from jax.experimental.pallas import tpu as pltpu
from jax.experimental.pallas import tpu_sc as plsc

NC = 2
NS = 16
NW = NC * NS

_SC_PARAMS = pltpu.CompilerParams()
if "needs_layout_passes" in pltpu.CompilerParams.__dataclass_fields__:
    _SC_PARAMS = dataclasses.replace(_SC_PARAMS, needs_layout_passes=False)

_LEAK = 1.0 / 5.5
_EPS = 1e-5


def _sc_scatter(edge_feat, dst, src, n_padded):
    E, D = edge_feat.shape
    Np = n_padded
    ch = 128
    gtot = E // ch
    nchunk = (gtot + NW - 1) // NW
    rows = Np // NS
    mesh = plsc.VectorSubcoreMesh(core_axis_name="c", subcore_axis_name="s")

    @functools.partial(
        pl.kernel,
        out_type=(
            jax.ShapeDtypeStruct((NC, Np, D), jnp.float32),
            jax.ShapeDtypeStruct((NW * Np,), jnp.float32),
        ),
        mesh=mesh,
        scratch_types=[
            pltpu.VMEM_SHARED((Np, D), jnp.float32),
            pltpu.VMEM((ch, D), jnp.float32),
            pltpu.VMEM((1, ch), jnp.int32),
            pltpu.VMEM((1, ch), jnp.int32),
            pltpu.VMEM((Np,), jnp.float32),
        ],
    )
    def k(ef_hbm, dst_hbm, src_hbm, seg_out, deg_out,
          seg_sh, buf, idxd, idxs, hist):
        c = lax.axis_index("c")
        s = lax.axis_index("s")
        wid = s * NC + c

        @pl.loop(0, ch)
        def _(i):
            @pl.loop(0, D // 16)
            def _(j):
                buf[i, pl.ds(j * 16, 16)] = jnp.zeros((16,), jnp.float32)

        @pl.loop(0, Np // 16)
        def _(i):
            hist[pl.ds(i * 16, 16)] = jnp.zeros((16,), jnp.float32)

        @pl.loop(0, rows // ch)
        def _(r):
            pltpu.sync_copy(buf, seg_sh.at[pl.ds(s * rows + r * ch, ch)])

        plsc.subcore_barrier()

        @pl.loop(0, nchunk)
        def _(t):
            g = t * NW + wid

            @pl.when(g < gtot)
            def _():
                base = g * ch
                pltpu.sync_copy(dst_hbm.at[pl.ds(base, ch)], idxd.at[0])
                pltpu.sync_copy(ef_hbm.at[pl.ds(base, ch)], buf)
                pltpu.sync_copy(buf, seg_sh.at[idxd.at[0]], add=True)

        plsc.subcore_barrier()

        @pl.loop(0, rows // ch)
        def _(r):
            off = s * rows + r * ch
            pltpu.sync_copy(seg_sh.at[pl.ds(off, ch)], buf)
            pltpu.sync_copy(buf, seg_out.at[c, pl.ds(off, ch)])

        pltpu.sync_copy(hist, deg_out.at[pl.ds(wid * Np, Np)])

    return k(edge_feat, dst, src)


def _sc_gather(pd, coef, ps, dst, src):
    N, D = pd.shape
    E = dst.shape[0]
    ept = E // NW
    ch = 80
    nchunk = ept // ch
    mesh = plsc.VectorSubcoreMesh(core_axis_name="c", subcore_axis_name="s")

    @functools.partial(
        pl.kernel,
        out_type=(
            jax.ShapeDtypeStruct((E, D), jnp.float32),
            jax.ShapeDtypeStruct((E,), jnp.float32),
            jax.ShapeDtypeStruct((E, D), jnp.float32),
        ),
        mesh=mesh,
        compiler_params=_SC_PARAMS,
        scratch_types=[
            pltpu.VMEM((ch,), jnp.int32),
            pltpu.VMEM((ch,), jnp.int32),
            pltpu.VMEM((ch, D), jnp.float32),
            pltpu.VMEM((ch,), jnp.float32),
            pltpu.VMEM((ch, D), jnp.float32),
            pltpu.VMEM((N,), jnp.float32),
            pltpu.SemaphoreType.DMA,
            pltpu.SemaphoreType.DMA,
        ],
    )
    def k(pd_hbm, coef_hbm, ps_hbm, dst_hbm, src_hbm, dg_out, ce_out, sg_out,
          idxd, idxs, bufd, bufc, bufs, ctab, semd, sems):
        c = lax.axis_index("c")
        s = lax.axis_index("s")
        wid = s * NC + c
        pltpu.sync_copy(coef_hbm, ctab)

        @pl.loop(0, nchunk)
        def _(t):
            base = wid * ept + t * ch
            pltpu.sync_copy(dst_hbm.at[pl.ds(base, ch)], idxd)
            pltpu.sync_copy(src_hbm.at[pl.ds(base, ch)], idxs)
            cp1 = pltpu.async_copy(pd_hbm.at[idxd], bufd, semd)
            cp3 = pltpu.async_copy(ps_hbm.at[idxs], bufs, sems)

            @pl.loop(0, ch // 16)
            def _(i):
                iv = idxd[pl.ds(i * 16, 16)]
                bufc[pl.ds(i * 16, 16)] = plsc.load_gather(ctab, [iv])

            cp1.wait()
            cp3.wait()
            pltpu.sync_copy(bufd, dg_out.at[pl.ds(base, ch)])
            pltpu.sync_copy(bufc, ce_out.at[pl.ds(base, ch)])
            pltpu.sync_copy(bufs, sg_out.at[pl.ds(base, ch)])

    return k(pd, coef, ps, dst, src)


def _full(shape):
    return pl.BlockSpec(shape, lambda i: (0,) * len(shape))


def _node_pass1(nf, seg0, seg1, degc, in_w, nloop_w, src_w, dst_w,
                nbias2, nw1, nb12):
    N, D = nf.shape
    nb = 2000
    grid = N // nb

    def body(nf_r, seg0_r, seg1_r, deg_r, in_w_r, nloop_r, src_w_r,
             dst_w_r, nbias_r, nw1_r, nb1_r,
             h_o, ps_o, pd_o, ce_o, s1_o, s2_o):
        segs = seg0_r[...] + seg1_r[...]
        pre = (jnp.dot(nf_r[...], nloop_r[...], preferred_element_type=jnp.float32)
               - jnp.dot(segs, in_w_r[...], preferred_element_type=jnp.float32)
               + nbias_r[...])
        h = jnp.dot(pre, nw1_r[...], preferred_element_type=jnp.float32) + nb1_r[...]
        h_o[...] = h
        ps_o[...] = jnp.dot(nf_r[...], src_w_r[...], preferred_element_type=jnp.float32)
        pd_o[...] = jnp.dot(nf_r[...], dst_w_r[...], preferred_element_type=jnp.float32)
        ce_o[...] = 2.0 * (1.0 + jnp.log2(1.0 + deg_r[...]))
        p1 = jnp.sum(h, axis=0, keepdims=True)
        p2 = jnp.sum(h * h, axis=0, keepdims=True)

        @pl.when(pl.program_id(0) == 0)
        def _():
            s1_o[...] = p1
            s2_o[...] = p2

        @pl.when(pl.program_id(0) > 0)
        def _():
            s1_o[...] += p1
            s2_o[...] += p2

    blk = lambda *shape: pl.BlockSpec(shape, lambda i: (i,) + (0,) * (len(shape) - 1))
    return pl.pallas_call(
        body,
        grid=(grid,),
        in_specs=[blk(nb, D), blk(nb, D), blk(nb, D), blk(nb, 1),
                  _full((D, D)), _full((D, D)), _full((D, D)), _full((D, D)),
                  _full((1, D)), _full((D, D)), _full((1, D))],
        out_specs=[blk(nb, D), blk(nb, D), blk(nb, D), blk(nb, 1),
                   _full((1, D)), _full((1, D))],
        out_shape=[
            jax.ShapeDtypeStruct((N, D), jnp.float32),
            jax.ShapeDtypeStruct((N, D), jnp.float32),
            jax.ShapeDtypeStruct((N, D), jnp.float32),
            jax.ShapeDtypeStruct((N, 1), jnp.float32),
            jax.ShapeDtypeStruct((1, D), jnp.float32),
            jax.ShapeDtypeStruct((1, D), jnp.float32),
        ],
    )(nf, seg0, seg1, degc, in_w, nloop_w, src_w, dst_w, nbias2, nw1, nb12)


def _edge_pass1(ef, dg, ce, sg, eloop_w, src_w, dst_w, ebias2, ew1, eb12):
    E, D = ef.shape
    be = 1280
    grid = E // be

    def body(ef_r, dg_r, ce_r, sg_r, eloop_r, src_w_r, dst_w_r, ebias_r,
             ew1_r, eb1_r, h_o, s1_o, s2_o):
        a = jnp.dot(ef_r[...], eloop_r[...], preferred_element_type=jnp.float32)
        b = jnp.dot(ef_r[...], src_w_r[...] - dst_w_r[...],
                    preferred_element_type=jnp.float32)
        pre = a + ce_r[...] * b + dg_r[...] - sg_r[...] + ebias_r[...]
        h = jnp.dot(pre, ew1_r[...], preferred_element_type=jnp.float32) + eb1_r[...]
        h_o[...] = h
        p1 = jnp.sum(h, axis=0, keepdims=True)
        p2 = jnp.sum(h * h, axis=0, keepdims=True)

        @pl.when(pl.program_id(0) == 0)
        def _():
            s1_o[...] = p1
            s2_o[...] = p2

        @pl.when(pl.program_id(0) > 0)
        def _():
            s1_o[...] += p1
            s2_o[...] += p2

    blk = lambda *shape: pl.BlockSpec(shape, lambda i: (i,) + (0,) * (len(shape) - 1))
    return pl.pallas_call(
        body,
        grid=(grid,),
        in_specs=[blk(be, D), blk(be, D), blk(be, 1), blk(be, D),
                  _full((D, D)), _full((D, D)), _full((D, D)),
                  _full((1, D)), _full((D, D)), _full((1, D))],
        out_specs=[blk(be, D), _full((1, D)), _full((1, D))],
        out_shape=[
            jax.ShapeDtypeStruct((E, D), jnp.float32),
            jax.ShapeDtypeStruct((1, D), jnp.float32),
            jax.ShapeDtypeStruct((1, D), jnp.float32),
        ],
    )(ef, dg, ce, sg, eloop_w, src_w, dst_w, ebias2, ew1, eb12)


def _pass2(h, scale, shift, w2, b22, rows_per_block):
    M, D = h.shape
    grid = M // rows_per_block

    def body(h_r, scale_r, shift_r, w2_r, b2_r, o_r):
        x = h_r[...] * scale_r[...] + shift_r[...]
        x = jnp.where(x >= 0, x, x * _LEAK)
        o_r[...] = jnp.dot(x, w2_r[...], preferred_element_type=jnp.float32) + b2_r[...]

    blk = lambda *shape: pl.BlockSpec(shape, lambda i: (i,) + (0,) * (len(shape) - 1))
    return pl.pallas_call(
        body,
        grid=(grid,),
        in_specs=[blk(rows_per_block, D), _full((1, D)), _full((1, D)),
                  _full((D, D)), _full((1, D))],
        out_specs=blk(rows_per_block, D),
        out_shape=jax.ShapeDtypeStruct((M, D), jnp.float32),
    )(h, scale, shift, w2, b22)


def _bn_coeffs(s1, s2, gamma, beta, count):
    mu = s1 / count
    var = s2 / count - mu * mu
    scale = gamma.reshape(1, -1) / jnp.sqrt(var + _EPS)
    shift = beta.reshape(1, -1) - mu * scale
    return scale, shift


def kernel(node_feat, edge_feat, edge_index, in_w, out_w, src_w, dst_w,
           nloop_w, eloop_w, nbias, ebias,
           nw1, nb1, ngamma, nbeta, nw2, nb2,
           ew1, eb1, egamma, ebeta, ew2, eb2):
    N, D = node_feat.shape
    E = edge_feat.shape[0]
    src = edge_index[0]
    dst = edge_index[1]

    seg = jax.ops.segment_sum(edge_feat, dst, num_segments=N)
    zseg = jnp.zeros((N, D), jnp.float32)
    degc = jax.ops.segment_sum(jnp.ones((E,), jnp.float32), src, num_segments=N).reshape(N, 1)

    h_n, ps, pd, coef, ns1, ns2 = _node_pass1(
        node_feat, seg, zseg, degc,
        in_w, nloop_w, src_w, dst_w,
        nbias.reshape(1, D), nw1, nb1.reshape(1, D))

    nscale, nshift = _bn_coeffs(ns1, ns2, ngamma, nbeta, float(N))
    nout = _pass2(h_n, nscale, nshift, nw2, nb2.reshape(1, D), 2000)

    dg, ce, sg = _sc_gather(pd, coef[:, 0], ps, dst, src)

    h_e, es1, es2 = _edge_pass1(edge_feat, dg, ce.reshape(E, 1), sg,
                                eloop_w, src_w, dst_w,
                                ebias.reshape(1, D), ew1, eb1.reshape(1, D))
    escale, eshift = _bn_coeffs(es1, es2, egamma, ebeta, float(E))
    eout = _pass2(h_e, escale, eshift, ew2, eb2.reshape(1, D), 1280)

    return (nout, eout)
